# baseline (speedup 1.0000x reference)
import jax
import jax.numpy as jnp
from jax.experimental import pallas as pl
from jax.experimental.pallas import tpu as pltpu

_NBUF = 4

def _pipeline_body(x_hbm, o_hbm, in_buf, in_sem, out_sem):
    b = x_hbm.shape[0]
    def in_copy(i, slot):
        return pltpu.make_async_copy(x_hbm.at[i], in_buf.at[slot], in_sem.at[slot])
    def out_copy(i, slot):
        return pltpu.make_async_copy(in_buf.at[slot], o_hbm.at[i], out_sem.at[slot])
    for s in range(_NBUF):
        in_copy(s, s).start()
    for i in range(b):
        slot = i % _NBUF
        in_copy(i, slot).wait()
        out_copy(i, slot).start()
        nxt = i + _NBUF
        if nxt < b:
            out_copy(i, slot).wait()
            in_copy(nxt, slot).start()
    for i in range(b - _NBUF, b):
        out_copy(i, i % _NBUF).wait()

def kernel(input):
    b, e, h, w = input.shape
    hw = h * w
    r, c = 432, 1024
    x = input.reshape(b, r, c)
    out = pl.pallas_call(
        _pipeline_body,
        in_specs=[pl.BlockSpec(memory_space=pltpu.MemorySpace.HBM)],
        out_specs=pl.BlockSpec(memory_space=pltpu.MemorySpace.HBM),
        out_shape=jax.ShapeDtypeStruct((b, r, c), x.dtype),
        scratch_shapes=[
            pltpu.VMEM((_NBUF, r, c), x.dtype),
            pltpu.SemaphoreType.DMA((_NBUF,)),
            pltpu.SemaphoreType.DMA((_NBUF,)),
        ],
    )(x)
    length = jnp.full((b,), True, dtype=bool)
    return (out, length)


# speedup vs baseline: 4.5127x; 4.5127x over previous
import jax
import jax.numpy as jnp
from jax.experimental import pallas as pl
from jax.experimental.pallas import tpu as pltpu

_NBUF = 8


def _pipeline_body(x_hbm, o_hbm, in_buf, out_buf, in_sem, out_sem):
    b = x_hbm.shape[0]

    def in_copy(i, slot):
        return pltpu.make_async_copy(x_hbm.at[i], in_buf.at[slot], in_sem.at[slot])

    def out_copy(i, slot):
        return pltpu.make_async_copy(out_buf.at[slot], o_hbm.at[i], out_sem.at[slot])

    for s in range(_NBUF):
        in_copy(s, s).start()
    for i in range(b):
        slot = i % _NBUF
        in_copy(i, slot).wait()
        if i >= _NBUF:
            out_copy(i - _NBUF, slot).wait()
        out_buf[slot] = in_buf[slot].T
        out_copy(i, slot).start()
        nxt = i + _NBUF
        if nxt < b:
            in_copy(nxt, slot).start()
    for i in range(b - _NBUF, b):
        out_copy(i, i % _NBUF).wait()


def kernel(input):
    b, e, h, w = input.shape
    hw = h * w
    x = input.reshape(b, e, hw)
    out = pl.pallas_call(
        _pipeline_body,
        in_specs=[pl.BlockSpec(memory_space=pltpu.MemorySpace.HBM)],
        out_specs=pl.BlockSpec(memory_space=pltpu.MemorySpace.HBM),
        out_shape=jax.ShapeDtypeStruct((b, hw, e), x.dtype),
        scratch_shapes=[
            pltpu.VMEM((_NBUF, e, hw), x.dtype),
            pltpu.VMEM((_NBUF, hw, e), x.dtype),
            pltpu.SemaphoreType.DMA((_NBUF,)),
            pltpu.SemaphoreType.DMA((_NBUF,)),
        ],
    )(x)
    length = jnp.full((b,), True, dtype=bool)
    return (out, length)


# speedup vs baseline: 4.5430x; 1.0067x over previous
import jax
import jax.numpy as jnp
from jax.experimental import pallas as pl
from jax.experimental.pallas import tpu as pltpu

_NBUF = 4
_SPLIT = 512


def _pipeline_body(x_hbm, o_hbm, in_buf1, in_buf2, out_buf, in_sem1, in_sem2, out_sem):
    b = x_hbm.shape[0]

    def in_copy1(i, slot):
        return pltpu.make_async_copy(
            x_hbm.at[i, :, 0:_SPLIT], in_buf1.at[slot], in_sem1.at[slot]
        )

    def in_copy2(i, slot):
        return pltpu.make_async_copy(
            x_hbm.at[i, :, _SPLIT:], in_buf2.at[slot], in_sem2.at[slot]
        )

    def out_copy(i, slot):
        return pltpu.make_async_copy(out_buf.at[slot], o_hbm.at[i], out_sem.at[slot])

    for s in range(_NBUF):
        in_copy1(s, s).start()
        in_copy2(s, s).start()
    for i in range(b):
        slot = i % _NBUF
        in_copy1(i, slot).wait()
        in_copy2(i, slot).wait()
        if i >= _NBUF:
            out_copy(i - _NBUF, slot).wait()
        out_buf[slot, 0:_SPLIT] = in_buf1[slot].T
        out_buf[slot, _SPLIT:] = in_buf2[slot].T
        out_copy(i, slot).start()
        nxt = i + _NBUF
        if nxt < b:
            in_copy1(nxt, slot).start()
            in_copy2(nxt, slot).start()
    for i in range(b - _NBUF, b):
        out_copy(i, i % _NBUF).wait()


def kernel(input):
    b, e, h, w = input.shape
    hw = h * w
    x = input.reshape(b, e, hw)
    out = pl.pallas_call(
        _pipeline_body,
        in_specs=[pl.BlockSpec(memory_space=pltpu.MemorySpace.HBM)],
        out_specs=pl.BlockSpec(memory_space=pltpu.MemorySpace.HBM),
        out_shape=jax.ShapeDtypeStruct((b, hw, e), x.dtype),
        scratch_shapes=[
            pltpu.VMEM((_NBUF, e, _SPLIT), x.dtype),
            pltpu.VMEM((_NBUF, e, hw - _SPLIT), x.dtype),
            pltpu.VMEM((_NBUF, hw, e), x.dtype),
            pltpu.SemaphoreType.DMA((_NBUF,)),
            pltpu.SemaphoreType.DMA((_NBUF,)),
            pltpu.SemaphoreType.DMA((_NBUF,)),
        ],
    )(x)
    length = jnp.full((b,), True, dtype=bool)
    return (out, length)
